# initial kernel scaffold (unmeasured)
import jax
import jax.numpy as jnp
from jax import lax
from jax.experimental import pallas as pl
from jax.experimental.pallas import tpu as pltpu

N_DEV = 4
NT = 512


def kernel(x, w_mat, scale_x, scale_w):
    m_total, k_sh = x.shape
    k_total, n_total = w_mat.shape
    m_per = m_total // N_DEV
    n_tiles = n_total // NT

    def body(x_ref, w_ref, sx_ref, sw_ref, out_ref,
             xq_ref, comm_ref, commb_ref, send_sems, recv_sems):
        me = lax.axis_index("i")
        step = pl.program_id(0)

        @pl.when(step == 0)
        def _comm():
            barrier = pltpu.get_barrier_semaphore()
            for off in (1, 2, 3):
                pl.semaphore_signal(
                    barrier, inc=1,
                    device_id=((me + off) % N_DEV,),
                    device_id_type=pl.DeviceIdType.MESH,
                )
            pl.semaphore_wait(barrier, N_DEV - 1)

            xq_ref[...] = x_ref[...].astype(jnp.float8_e5m2)

            rdmas = []
            for off in (1, 2, 3):
                p = (me + off) % N_DEV
                r = pltpu.make_async_remote_copy(
                    src_ref=xq_ref.at[pl.ds(p * m_per, m_per), :],
                    dst_ref=comm_ref.at[me],
                    send_sem=send_sems.at[off - 1],
                    recv_sem=recv_sems.at[me],
                    device_id=(p,),
                    device_id_type=pl.DeviceIdType.MESH,
                )
                r.start()
                rdmas.append(r)

            commb_ref[me] = x_ref[pl.ds(me * m_per, m_per), :].astype(jnp.bfloat16)

            for off in (1, 3, 2):
                j = (me + off) % N_DEV
                recv = pltpu.make_async_remote_copy(
                    src_ref=comm_ref.at[j],
                    dst_ref=comm_ref.at[j],
                    send_sem=send_sems.at[0],
                    recv_sem=recv_sems.at[j],
                    device_id=(me,),
                    device_id_type=pl.DeviceIdType.MESH,
                )
                recv.wait_recv()
                commb_ref[j] = comm_ref[j].astype(jnp.bfloat16)

            for r in rdmas:
                r.wait_send()

        s = sx_ref[0] * sw_ref[0]
        wt = w_ref[...]
        acc = jnp.zeros((m_per, NT), jnp.float32)
        for j in range(N_DEV):
            wj = wt[j * k_sh:(j + 1) * k_sh, :].astype(jnp.bfloat16)
            acc = acc + lax.dot_general(
                commb_ref[j], wj,
                (((1,), (0,)), ((), ())),
                preferred_element_type=jnp.float32,
            )
        out_ref[...] = jnp.maximum(acc * s, 0.0)

    return pl.pallas_call(
        body,
        grid=(n_tiles,),
        in_specs=[
            pl.BlockSpec((m_total, k_sh), lambda i: (0, 0)),
            pl.BlockSpec((k_total, NT), lambda i: (0, i)),
            pl.BlockSpec(memory_space=pltpu.SMEM),
            pl.BlockSpec(memory_space=pltpu.SMEM),
        ],
        out_specs=pl.BlockSpec((m_per, NT), lambda i: (0, i)),
        out_shape=jax.ShapeDtypeStruct((m_per, n_total), jnp.float32),
        scratch_shapes=[
            pltpu.VMEM((m_total, k_sh), jnp.float8_e5m2),
            pltpu.VMEM((N_DEV, m_per, k_sh), jnp.float8_e5m2),
            pltpu.VMEM((N_DEV, m_per, k_sh), jnp.bfloat16),
            pltpu.SemaphoreType.DMA((3,)),
            pltpu.SemaphoreType.DMA((N_DEV,)),
        ],
        compiler_params=pltpu.CompilerParams(
            dimension_semantics=("arbitrary",),
            collective_id=0,
        ),
    )(x, w_mat, scale_x, scale_w)


# baseline (device time: 136463 ns/iter reference)
import jax
import jax.numpy as jnp
from jax import lax
from jax.experimental import pallas as pl
from jax.experimental.pallas import tpu as pltpu

N_DEV = 4
NT = 512


def kernel(x, w_mat, scale_x, scale_w):
    m_total, k_sh = x.shape
    k_total, n_total = w_mat.shape
    m_per = m_total // N_DEV
    n_tiles = n_total // NT

    def body(x_ref, w_ref, sx_ref, sw_ref, out_ref,
             xq_ref, comm_ref, commb_ref, send_sems, recv_sems):
        me = lax.axis_index("i")
        step = pl.program_id(0)

        @pl.when(step == 0)
        def _comm():
            barrier = pltpu.get_barrier_semaphore()
            for off in (1, 2, 3):
                pl.semaphore_signal(
                    barrier, inc=1,
                    device_id=((me + off) % N_DEV,),
                    device_id_type=pl.DeviceIdType.MESH,
                )
            pl.semaphore_wait(barrier, N_DEV - 1)

            xq_ref[...] = x_ref[...].astype(jnp.float8_e5m2)

            rdmas = []
            for off in (1, 2, 3):
                p = (me + off) % N_DEV
                r = pltpu.make_async_remote_copy(
                    src_ref=xq_ref.at[pl.ds(p * m_per, m_per), :],
                    dst_ref=comm_ref.at[me],
                    send_sem=send_sems.at[off - 1],
                    recv_sem=recv_sems.at[me],
                    device_id=(p,),
                    device_id_type=pl.DeviceIdType.MESH,
                )
                r.start()
                rdmas.append(r)

            commb_ref[me] = x_ref[pl.ds(me * m_per, m_per), :].astype(jnp.bfloat16)

            for off in (1, 3, 2):
                j = (me + off) % N_DEV
                recv = pltpu.make_async_remote_copy(
                    src_ref=comm_ref.at[j],
                    dst_ref=comm_ref.at[j],
                    send_sem=send_sems.at[0],
                    recv_sem=recv_sems.at[j],
                    device_id=(me,),
                    device_id_type=pl.DeviceIdType.MESH,
                )
                recv.wait_recv()
                commb_ref[j] = comm_ref[j].astype(jnp.bfloat16)

            for r in rdmas:
                r.wait_send()

        s = sx_ref[0] * sw_ref[0]
        wt = w_ref[...]
        acc = jnp.zeros((m_per, NT), jnp.float32)
        for j in range(N_DEV):
            wj = wt[j * k_sh:(j + 1) * k_sh, :].astype(jnp.bfloat16)
            acc = acc + lax.dot_general(
                commb_ref[j], wj,
                (((1,), (0,)), ((), ())),
                preferred_element_type=jnp.float32,
            )
        out_ref[...] = jnp.maximum(acc * s, 0.0)

    return pl.pallas_call(
        body,
        grid=(n_tiles,),
        in_specs=[
            pl.BlockSpec((m_total, k_sh), lambda i: (0, 0)),
            pl.BlockSpec((k_total, NT), lambda i: (0, i)),
            pl.BlockSpec(memory_space=pltpu.SMEM),
            pl.BlockSpec(memory_space=pltpu.SMEM),
        ],
        out_specs=pl.BlockSpec((m_per, NT), lambda i: (0, i)),
        out_shape=jax.ShapeDtypeStruct((m_per, n_total), jnp.float32),
        scratch_shapes=[
            pltpu.VMEM((m_total, k_sh), jnp.float8_e5m2),
            pltpu.VMEM((N_DEV, m_per, k_sh), jnp.float8_e5m2),
            pltpu.VMEM((N_DEV, m_per, k_sh), jnp.bfloat16),
            pltpu.SemaphoreType.DMA((3,)),
            pltpu.SemaphoreType.DMA((N_DEV,)),
        ],
        compiler_params=pltpu.CompilerParams(
            dimension_semantics=("arbitrary",),
            collective_id=0,
            vmem_limit_bytes=100 * 1024 * 1024,
        ),
    )(x, w_mat, scale_x, scale_w)


# device time: 123276 ns/iter; 1.1070x vs baseline; 1.1070x over previous
import jax
import jax.numpy as jnp
from jax import lax
from jax.experimental import pallas as pl
from jax.experimental.pallas import tpu as pltpu

N_DEV = 4
NT = 1024


def kernel(x, w_mat, scale_x, scale_w):
    m_total, k_sh = x.shape
    k_total, n_total = w_mat.shape
    m_per = m_total // N_DEV
    n_tiles = n_total // NT

    me_out = lax.axis_index("i")
    perm = (me_out + jnp.array([0, 1, 3, 2], jnp.int32)) % N_DEV

    xq = x.astype(jnp.float8_e5m2)

    def body(perm_ref, xq_ref, w_ref, sx_ref, sw_ref, out_ref,
             comm_ref, send_sems, recv_sems):
        me = lax.axis_index("i")
        kb = pl.program_id(0)
        nt = pl.program_id(1)

        @pl.when((kb == 0) & (nt == 0))
        def _start():
            barrier = pltpu.get_barrier_semaphore()
            for off in (1, 2, 3):
                pl.semaphore_signal(
                    barrier, inc=1,
                    device_id=((me + off) % N_DEV,),
                    device_id_type=pl.DeviceIdType.MESH,
                )
            pl.semaphore_wait(barrier, N_DEV - 1)

            comm_ref[me] = xq_ref[pl.ds(me * m_per, m_per), :]

            for off in (1, 2, 3):
                p = (me + off) % N_DEV
                pltpu.make_async_remote_copy(
                    src_ref=xq_ref.at[pl.ds(p * m_per, m_per), :],
                    dst_ref=comm_ref.at[me],
                    send_sem=send_sems.at[off - 1],
                    recv_sem=recv_sems.at[me],
                    device_id=(p,),
                    device_id_type=pl.DeviceIdType.MESH,
                ).start()

        @pl.when((kb > 0) & (nt == 0))
        def _wait_block():
            j = perm_ref[kb]
            pltpu.make_async_remote_copy(
                src_ref=comm_ref.at[j],
                dst_ref=comm_ref.at[j],
                send_sem=send_sems.at[0],
                recv_sem=recv_sems.at[j],
                device_id=(me,),
                device_id_type=pl.DeviceIdType.MESH,
            ).wait_recv()

        j = perm_ref[kb]
        wq = w_ref[...].astype(jnp.float8_e5m2)
        part = lax.dot_general(
            comm_ref[j], wq,
            (((1,), (0,)), ((), ())),
            preferred_element_type=jnp.float32,
        )
        ncols = pl.ds(nt * NT, NT)

        @pl.when(kb == 0)
        def _init():
            out_ref[:, ncols] = part

        @pl.when((kb > 0) & (kb < N_DEV - 1))
        def _acc():
            out_ref[:, ncols] += part

        @pl.when(kb == N_DEV - 1)
        def _final():
            s = sx_ref[0] * sw_ref[0]
            out_ref[:, ncols] = jnp.maximum((out_ref[:, ncols] + part) * s, 0.0)

        @pl.when((kb == N_DEV - 1) & (nt == n_tiles - 1))
        def _drain_sends():
            for off in (1, 2, 3):
                p = (me + off) % N_DEV
                pltpu.make_async_remote_copy(
                    src_ref=xq_ref.at[pl.ds(p * m_per, m_per), :],
                    dst_ref=comm_ref.at[me],
                    send_sem=send_sems.at[off - 1],
                    recv_sem=recv_sems.at[me],
                    device_id=(p,),
                    device_id_type=pl.DeviceIdType.MESH,
                ).wait_send()

    grid_spec = pltpu.PrefetchScalarGridSpec(
        num_scalar_prefetch=1,
        grid=(N_DEV, n_tiles),
        in_specs=[
            pl.BlockSpec((m_total, k_sh), lambda kb, nt, p: (0, 0)),
            pl.BlockSpec((k_sh, NT), lambda kb, nt, p: (p[kb], nt)),
            pl.BlockSpec(memory_space=pltpu.SMEM),
            pl.BlockSpec(memory_space=pltpu.SMEM),
        ],
        out_specs=pl.BlockSpec((m_per, n_total), lambda kb, nt, p: (0, 0)),
        scratch_shapes=[
            pltpu.VMEM((N_DEV, m_per, k_sh), jnp.float8_e5m2),
            pltpu.SemaphoreType.DMA((3,)),
            pltpu.SemaphoreType.DMA((N_DEV,)),
        ],
    )

    return pl.pallas_call(
        body,
        grid_spec=grid_spec,
        out_shape=jax.ShapeDtypeStruct((m_per, n_total), jnp.float32),
        compiler_params=pltpu.CompilerParams(
            dimension_semantics=("arbitrary", "arbitrary"),
            collective_id=0,
            vmem_limit_bytes=100 * 1024 * 1024,
        ),
    )(perm, xq, w_mat, scale_x, scale_w)


# device time: 121232 ns/iter; 1.1256x vs baseline; 1.0169x over previous
import jax
import jax.numpy as jnp
from jax import lax
from jax.experimental import pallas as pl
from jax.experimental.pallas import tpu as pltpu

N_DEV = 4
NT = 1024
C = 4


def kernel(x, w_mat, scale_x, scale_w):
    m_total, k_sh = x.shape
    k_total, n_total = w_mat.shape
    m_per = m_total // N_DEV
    n_tiles = n_total // NT
    m_c = m_per // C

    me_out = lax.axis_index("i")
    perm = (me_out + jnp.array([0, 1, 3, 2], jnp.int32)) % N_DEV

    xq = x.astype(jnp.float8_e5m2)

    def body(perm_ref, xq_ref, w_ref, sx_ref, sw_ref, out_ref,
             comm_ref, send_sems, recv_sems):
        me = lax.axis_index("i")
        kb = pl.program_id(0)
        nt = pl.program_id(1)

        def send_chunk(off_i, off, c):
            p = (me + off) % N_DEV
            return pltpu.make_async_remote_copy(
                src_ref=xq_ref.at[pl.ds(p * m_per + c * m_c, m_c), :],
                dst_ref=comm_ref.at[me, pl.ds(c * m_c, m_c), :],
                send_sem=send_sems.at[off_i, c],
                recv_sem=recv_sems.at[me, c],
                device_id=(p,),
                device_id_type=pl.DeviceIdType.MESH,
            )

        @pl.when((kb == 0) & (nt == 0))
        def _start():
            barrier = pltpu.get_barrier_semaphore()
            for off in (1, 2, 3):
                pl.semaphore_signal(
                    barrier, inc=1,
                    device_id=((me + off) % N_DEV,),
                    device_id_type=pl.DeviceIdType.MESH,
                )
            pl.semaphore_wait(barrier, N_DEV - 1)

            comm_ref[me] = xq_ref[pl.ds(me * m_per, m_per), :]

            for c in range(C):
                send_chunk(0, 1, c).start()
                send_chunk(1, 3, c).start()
            for c in range(C):
                send_chunk(2, 2, c).start()

        j = perm_ref[kb]
        wq = w_ref[...].astype(jnp.float8_e5m2)
        dims = (((1,), (0,)), ((), ()))
        s = sx_ref[0] * sw_ref[0]
        ncols = pl.ds(nt * NT, NT)

        def accum(rows, part):
            @pl.when(kb == 0)
            def _init():
                out_ref[rows, ncols] = part

            @pl.when((kb > 0) & (kb < N_DEV - 1))
            def _acc():
                out_ref[rows, ncols] += part

            @pl.when(kb == N_DEV - 1)
            def _final():
                out_ref[rows, ncols] = jnp.maximum(
                    (out_ref[rows, ncols] + part) * s, 0.0)

        @pl.when(nt == 0)
        def _first_tile():
            for c in range(C):
                @pl.when(kb > 0)
                def _wait_chunk():
                    pltpu.make_async_remote_copy(
                        src_ref=comm_ref.at[j, pl.ds(c * m_c, m_c), :],
                        dst_ref=comm_ref.at[j, pl.ds(c * m_c, m_c), :],
                        send_sem=send_sems.at[0, c],
                        recv_sem=recv_sems.at[j, c],
                        device_id=(me,),
                        device_id_type=pl.DeviceIdType.MESH,
                    ).wait_recv()

                rows = pl.ds(c * m_c, m_c)
                part = lax.dot_general(
                    comm_ref[j, rows, :], wq, dims,
                    preferred_element_type=jnp.float32,
                )
                accum(rows, part)

        @pl.when(nt > 0)
        def _rest_tiles():
            part = lax.dot_general(
                comm_ref[j], wq, dims,
                preferred_element_type=jnp.float32,
            )
            accum(slice(None), part)

        @pl.when((kb == N_DEV - 1) & (nt == n_tiles - 1))
        def _drain_sends():
            for off_i, off in enumerate((1, 3, 2)):
                for c in range(C):
                    send_chunk(off_i, off, c).wait_send()

    grid_spec = pltpu.PrefetchScalarGridSpec(
        num_scalar_prefetch=1,
        grid=(N_DEV, n_tiles),
        in_specs=[
            pl.BlockSpec((m_total, k_sh), lambda kb, nt, p: (0, 0)),
            pl.BlockSpec((k_sh, NT), lambda kb, nt, p: (p[kb], nt)),
            pl.BlockSpec(memory_space=pltpu.SMEM),
            pl.BlockSpec(memory_space=pltpu.SMEM),
        ],
        out_specs=pl.BlockSpec((m_per, n_total), lambda kb, nt, p: (0, 0)),
        scratch_shapes=[
            pltpu.VMEM((N_DEV, m_per, k_sh), jnp.float8_e5m2),
            pltpu.SemaphoreType.DMA((3, C)),
            pltpu.SemaphoreType.DMA((N_DEV, C)),
        ],
    )

    return pl.pallas_call(
        body,
        grid_spec=grid_spec,
        out_shape=jax.ShapeDtypeStruct((m_per, n_total), jnp.float32),
        compiler_params=pltpu.CompilerParams(
            dimension_semantics=("arbitrary", "arbitrary"),
            collective_id=0,
            vmem_limit_bytes=100 * 1024 * 1024,
        ),
    )(perm, xq, w_mat, scale_x, scale_w)


# device time: 120603 ns/iter; 1.1315x vs baseline; 1.0052x over previous
import jax
import jax.numpy as jnp
from jax import lax
from jax.experimental import pallas as pl
from jax.experimental.pallas import tpu as pltpu

N_DEV = 4
NT = 1024
C = 4


def kernel(x, w_mat, scale_x, scale_w):
    m_total, k_sh = x.shape
    k_total, n_total = w_mat.shape
    m_per = m_total // N_DEV
    n_tiles = n_total // NT
    m_c = m_per // C

    me_out = lax.axis_index("i")
    perm = (me_out + jnp.array([0, 1, 3, 2], jnp.int32)) % N_DEV

    xq = x.astype(jnp.float8_e5m2)

    def body(perm_ref, xq_ref, w_ref, sx_ref, sw_ref, out_ref,
             comm_ref, send_sems, recv_sems):
        me = lax.axis_index("i")
        kb = pl.program_id(0)
        nt = pl.program_id(1)

        def send_chunk(off_i, off, c):
            p = (me + off) % N_DEV
            return pltpu.make_async_remote_copy(
                src_ref=xq_ref.at[pl.ds(p * m_per + c * m_c, m_c), :],
                dst_ref=comm_ref.at[me, pl.ds(c * m_c, m_c), :],
                send_sem=send_sems.at[off_i, c],
                recv_sem=recv_sems.at[me, c],
                device_id=(p,),
                device_id_type=pl.DeviceIdType.MESH,
            )

        @pl.when((kb == 0) & (nt == 0))
        def _start():
            barrier = pltpu.get_barrier_semaphore()
            for off in (1, 2, 3):
                pl.semaphore_signal(
                    barrier, inc=1,
                    device_id=((me + off) % N_DEV,),
                    device_id_type=pl.DeviceIdType.MESH,
                )
            pl.semaphore_wait(barrier, N_DEV - 1)

            comm_ref[me] = xq_ref[pl.ds(me * m_per, m_per), :]

            for c in range(C):
                send_chunk(0, 3, c).start()

        @pl.when((kb == 1) & (nt == 0))
        def _phase2():
            for c in range(C):
                send_chunk(1, 1, c).start()

        @pl.when((kb == 2) & (nt == 0))
        def _phase3():
            for c in range(C):
                send_chunk(2, 2, c).start()

        j = perm_ref[kb]
        wq = w_ref[...].astype(jnp.float8_e5m2)
        dims = (((1,), (0,)), ((), ()))
        s = sx_ref[0] * sw_ref[0]
        ncols = pl.ds(nt * NT, NT)

        def accum(rows, part):
            @pl.when(kb == 0)
            def _init():
                out_ref[rows, ncols] = part

            @pl.when((kb > 0) & (kb < N_DEV - 1))
            def _acc():
                out_ref[rows, ncols] += part

            @pl.when(kb == N_DEV - 1)
            def _final():
                out_ref[rows, ncols] = jnp.maximum(
                    (out_ref[rows, ncols] + part) * s, 0.0)

        @pl.when(nt == 0)
        def _first_tile():
            for c in range(C):
                @pl.when(kb > 0)
                def _wait_chunk():
                    pltpu.make_async_remote_copy(
                        src_ref=comm_ref.at[j, pl.ds(c * m_c, m_c), :],
                        dst_ref=comm_ref.at[j, pl.ds(c * m_c, m_c), :],
                        send_sem=send_sems.at[0, c],
                        recv_sem=recv_sems.at[j, c],
                        device_id=(me,),
                        device_id_type=pl.DeviceIdType.MESH,
                    ).wait_recv()

                rows = pl.ds(c * m_c, m_c)
                part = lax.dot_general(
                    comm_ref[j, rows, :], wq, dims,
                    preferred_element_type=jnp.float32,
                )
                accum(rows, part)

        @pl.when(nt > 0)
        def _rest_tiles():
            part = lax.dot_general(
                comm_ref[j], wq, dims,
                preferred_element_type=jnp.float32,
            )
            accum(slice(None), part)

        @pl.when((kb == N_DEV - 1) & (nt == n_tiles - 1))
        def _drain_sends():
            for off_i, off in enumerate((3, 1, 2)):
                for c in range(C):
                    send_chunk(off_i, off, c).wait_send()

    grid_spec = pltpu.PrefetchScalarGridSpec(
        num_scalar_prefetch=1,
        grid=(N_DEV, n_tiles),
        in_specs=[
            pl.BlockSpec((m_total, k_sh), lambda kb, nt, p: (0, 0)),
            pl.BlockSpec((k_sh, NT), lambda kb, nt, p: (p[kb], nt)),
            pl.BlockSpec(memory_space=pltpu.SMEM),
            pl.BlockSpec(memory_space=pltpu.SMEM),
        ],
        out_specs=pl.BlockSpec((m_per, n_total), lambda kb, nt, p: (0, 0)),
        scratch_shapes=[
            pltpu.VMEM((N_DEV, m_per, k_sh), jnp.float8_e5m2),
            pltpu.SemaphoreType.DMA((3, C)),
            pltpu.SemaphoreType.DMA((N_DEV, C)),
        ],
    )

    return pl.pallas_call(
        body,
        grid_spec=grid_spec,
        out_shape=jax.ShapeDtypeStruct((m_per, n_total), jnp.float32),
        compiler_params=pltpu.CompilerParams(
            dimension_semantics=("arbitrary", "arbitrary"),
            collective_id=0,
            vmem_limit_bytes=100 * 1024 * 1024,
        ),
    )(perm, xq, w_mat, scale_x, scale_w)
